# baseline (speedup 1.0000x reference)
import functools

import jax
import jax.numpy as jnp
from jax import lax
from jax.experimental import pallas as pl
from jax.experimental.pallas import tpu as pltpu
from jax.experimental.pallas import tpu_sc as plsc

_EPS = 1e-07
_NEG_RATIO = 3.0

_N = 16 * 512 * 512
_NC = 2
_NS = 16
_NW = _NC * _NS
_PER_W = _N // _NW
_CHUNK = 16384
_NCHUNKS = _PER_W // _CHUNK
_LANES = 16


def _sc_partials(pred, gt, mask):
    mesh = plsc.VectorSubcoreMesh(core_axis_name="c", subcore_axis_name="s")

    @functools.partial(
        pl.kernel,
        mesh=mesh,
        out_type=jax.ShapeDtypeStruct((_NW, 4 * _LANES), jnp.float32),
        scratch_types=[
            pltpu.VMEM((_CHUNK,), jnp.float32),
            pltpu.VMEM((_CHUNK,), jnp.float32),
            pltpu.VMEM((_CHUNK,), jnp.float32),
            pltpu.VMEM((4 * _LANES,), jnp.float32),
        ],
    )
    def body(p_hbm, g_hbm, m_hbm, out_hbm, pb, gb, mb, acc):
        wid = lax.axis_index("s") * _NC + lax.axis_index("c")
        base = wid * _PER_W

        def outer(ci, accs):
            off = base + ci * _CHUNK
            pltpu.sync_copy(p_hbm.at[pl.ds(off, _CHUNK)], pb)
            pltpu.sync_copy(g_hbm.at[pl.ds(off, _CHUNK)], gb)
            pltpu.sync_copy(m_hbm.at[pl.ds(off, _CHUNK)], mb)

            def inner(vi, accs2):
                a_m, a_gm, a_pm, a_pgm = accs2
                s = vi * _LANES
                p = pb[pl.ds(s, _LANES)]
                g = gb[pl.ds(s, _LANES)]
                m = mb[pl.ds(s, _LANES)]
                pm = p * m
                return (a_m + m, a_gm + g * m, a_pm + pm, a_pgm + pm * g)

            return lax.fori_loop(0, _CHUNK // _LANES, inner, accs)

        z = jnp.zeros((_LANES,), jnp.float32)
        a_m, a_gm, a_pm, a_pgm = lax.fori_loop(0, _NCHUNKS, outer, (z, z, z, z))
        acc[pl.ds(0, _LANES)] = a_m
        acc[pl.ds(_LANES, _LANES)] = a_gm
        acc[pl.ds(2 * _LANES, _LANES)] = a_pm
        acc[pl.ds(3 * _LANES, _LANES)] = a_pgm
        pltpu.sync_copy(acc, out_hbm.at[wid])

    return body(pred, gt, mask)


def _epilogue_body(part_ref, out_ref):
    x = part_ref[...]
    s_m = jnp.sum(x[:, 0:16])
    s_gm = jnp.sum(x[:, 16:32])
    s_pm = jnp.sum(x[:, 32:48])
    s_pgm = jnp.sum(x[:, 48:64])
    loss = 1.0 - 2.0 * s_pgm / (s_pm + s_gm + _EPS)
    pos = s_gm
    neg = jnp.minimum(s_m - s_gm, _NEG_RATIO * pos)
    balanced = loss * (pos + neg) / (pos + neg + _EPS)
    fallback = loss * pos / (pos + _EPS)
    out_ref[0, 0] = jnp.where(neg > 0.0, balanced, fallback)


@jax.jit
def kernel(pred, gt, mask):
    partials = _sc_partials(
        pred.reshape(-1), gt.reshape(-1), mask.reshape(-1)
    )
    out = pl.pallas_call(
        _epilogue_body,
        out_specs=pl.BlockSpec((1, 1), memory_space=pltpu.SMEM),
        out_shape=jax.ShapeDtypeStruct((1, 1), jnp.float32),
    )(partials)
    return out.reshape(())


# speedup vs baseline: 1.3816x; 1.3816x over previous
import functools

import jax
import jax.numpy as jnp
from jax import lax
from jax.experimental import pallas as pl
from jax.experimental.pallas import tpu as pltpu
from jax.experimental.pallas import tpu_sc as plsc

_EPS = 1e-07
_NEG_RATIO = 3.0

_N = 16 * 512 * 512
_NC = 2
_NS = 16
_NW = _NC * _NS
_PER_W = _N // _NW
_CHUNK = 16384
_NCHUNKS = _PER_W // _CHUNK
_LANES = 16


_UNROLL = 8


def _sc_partials(pred, gt, mask):
    mesh = plsc.VectorSubcoreMesh(core_axis_name="c", subcore_axis_name="s")
    buf = pltpu.VMEM((_CHUNK,), jnp.float32)

    @functools.partial(
        pl.kernel,
        mesh=mesh,
        out_type=jax.ShapeDtypeStruct((_NW, 4 * _LANES), jnp.float32),
        scratch_types=[buf] * 6
        + [pltpu.SemaphoreType.DMA] * 6
        + [pltpu.VMEM((4 * _LANES,), jnp.float32)],
    )
    def body(p_hbm, g_hbm, m_hbm, out_hbm,
             pb0, gb0, mb0, pb1, gb1, mb1,
             sp0, sg0, sm0, sp1, sg1, sm1, acc):
        wid = lax.axis_index("s") * _NC + lax.axis_index("c")
        base = wid * _PER_W
        hbm = (p_hbm, g_hbm, m_hbm)
        bufs = ((pb0, gb0, mb0), (pb1, gb1, mb1))
        sems = ((sp0, sg0, sm0), (sp1, sg1, sm1))

        def start(ci):
            off = base + ci * _CHUNK
            slot = ci % 2
            return [
                pltpu.async_copy(h.at[pl.ds(off, _CHUNK)], b, s)
                for h, b, s in zip(hbm, bufs[slot], sems[slot])
            ]

        def compute(slot, accs):
            pb, gb, mb = bufs[slot]

            def inner(t, accs2):
                a_m, a_gm, a_pm, a_pgm = accs2
                s0 = t * (_UNROLL * _LANES)
                for u in range(_UNROLL):
                    s = s0 + u * _LANES
                    p = pb[pl.ds(s, _LANES)]
                    g = gb[pl.ds(s, _LANES)]
                    m = mb[pl.ds(s, _LANES)]
                    pm = p * m
                    a_m = a_m + m
                    a_gm = a_gm + g * m
                    a_pm = a_pm + pm
                    a_pgm = a_pgm + pm * g
                return (a_m, a_gm, a_pm, a_pgm)

            return lax.fori_loop(
                0, _CHUNK // (_UNROLL * _LANES), inner, accs, unroll=False
            )

        z = jnp.zeros((_LANES,), jnp.float32)
        accs = (z, z, z, z)
        pending = start(0)
        for ci in range(_NCHUNKS):
            nxt = start(ci + 1) if ci + 1 < _NCHUNKS else None
            for d in pending:
                d.wait()
            accs = compute(ci % 2, accs)
            pending = nxt
        a_m, a_gm, a_pm, a_pgm = accs
        acc[pl.ds(0, _LANES)] = a_m
        acc[pl.ds(_LANES, _LANES)] = a_gm
        acc[pl.ds(2 * _LANES, _LANES)] = a_pm
        acc[pl.ds(3 * _LANES, _LANES)] = a_pgm
        pltpu.sync_copy(acc, out_hbm.at[wid])

    return body(pred, gt, mask)


def _epilogue_body(part_ref, out_ref):
    x = part_ref[...]
    s_m = jnp.sum(x[:, 0:16])
    s_gm = jnp.sum(x[:, 16:32])
    s_pm = jnp.sum(x[:, 32:48])
    s_pgm = jnp.sum(x[:, 48:64])
    loss = 1.0 - 2.0 * s_pgm / (s_pm + s_gm + _EPS)
    pos = s_gm
    neg = jnp.minimum(s_m - s_gm, _NEG_RATIO * pos)
    balanced = loss * (pos + neg) / (pos + neg + _EPS)
    fallback = loss * pos / (pos + _EPS)
    out_ref[0, 0] = jnp.where(neg > 0.0, balanced, fallback)


@jax.jit
def kernel(pred, gt, mask):
    partials = _sc_partials(
        pred.reshape(-1), gt.reshape(-1), mask.reshape(-1)
    )
    out = pl.pallas_call(
        _epilogue_body,
        out_specs=pl.BlockSpec((1, 1), memory_space=pltpu.SMEM),
        out_shape=jax.ShapeDtypeStruct((1, 1), jnp.float32),
    )(partials)
    return out.reshape(())


# speedup vs baseline: 2.3939x; 1.7328x over previous
import functools

import jax
import jax.numpy as jnp
from jax import lax
from jax.experimental import pallas as pl
from jax.experimental.pallas import tpu as pltpu
from jax.experimental.pallas import tpu_sc as plsc

_EPS = 1e-07
_NEG_RATIO = 3.0

_N = 16 * 512 * 512
_NC = 2
_NS = 16
_NW = _NC * _NS
_COLS = 512
_ROWS = _N // _COLS
_ROWS_W = _ROWS // _NW
_BAND = 32
_NCHUNKS = _ROWS_W // _BAND
_LANES = 16


def _sc_partials(pred, gt, mask):
    mesh = plsc.VectorSubcoreMesh(core_axis_name="c", subcore_axis_name="s")
    buf = pltpu.VMEM((_BAND, _COLS), jnp.float32)

    @functools.partial(
        pl.kernel,
        mesh=mesh,
        out_type=jax.ShapeDtypeStruct((_NW, 4 * _LANES), jnp.float32),
        scratch_types=[buf] * 6
        + [pltpu.SemaphoreType.DMA] * 6
        + [pltpu.VMEM((4 * _LANES,), jnp.float32)],
    )
    def body(p_hbm, g_hbm, m_hbm, out_hbm,
             pb0, gb0, mb0, pb1, gb1, mb1,
             sp0, sg0, sm0, sp1, sg1, sm1, acc):
        wid = lax.axis_index("s") * _NC + lax.axis_index("c")
        base = wid * _ROWS_W
        hbm = (p_hbm, g_hbm, m_hbm)
        bufs = ((pb0, gb0, mb0), (pb1, gb1, mb1))
        sems = ((sp0, sg0, sm0), (sp1, sg1, sm1))

        def start(ci):
            row0 = base + ci * _BAND
            slot = ci % 2
            return [
                pltpu.async_copy(h.at[pl.ds(row0, _BAND)], b, s)
                for h, b, s in zip(hbm, bufs[slot], sems[slot])
            ]

        def compute(slot, accs):
            pb, gb, mb = bufs[slot]

            def inner(r, accs2):
                a_m, a_gm, a_pm, a_pgm = accs2
                for cu in range(_COLS // _LANES):
                    c = cu * _LANES
                    p = pb[r, pl.ds(c, _LANES)]
                    g = gb[r, pl.ds(c, _LANES)]
                    m = mb[r, pl.ds(c, _LANES)]
                    pm = p * m
                    a_m = a_m + m
                    a_gm = a_gm + g * m
                    a_pm = a_pm + pm
                    a_pgm = a_pgm + pm * g
                return (a_m, a_gm, a_pm, a_pgm)

            return lax.fori_loop(0, _BAND, inner, accs, unroll=False)

        z = jnp.zeros((_LANES,), jnp.float32)
        accs = (z, z, z, z)
        pending = start(0)
        for ci in range(_NCHUNKS):
            nxt = start(ci + 1) if ci + 1 < _NCHUNKS else None
            for d in pending:
                d.wait()
            accs = compute(ci % 2, accs)
            pending = nxt
        a_m, a_gm, a_pm, a_pgm = accs
        acc[pl.ds(0, _LANES)] = a_m
        acc[pl.ds(_LANES, _LANES)] = a_gm
        acc[pl.ds(2 * _LANES, _LANES)] = a_pm
        acc[pl.ds(3 * _LANES, _LANES)] = a_pgm
        pltpu.sync_copy(acc, out_hbm.at[wid])

    return body(pred, gt, mask)


def _epilogue_body(part_ref, out_ref):
    x = part_ref[...]
    s_m = jnp.sum(x[:, 0:16])
    s_gm = jnp.sum(x[:, 16:32])
    s_pm = jnp.sum(x[:, 32:48])
    s_pgm = jnp.sum(x[:, 48:64])
    loss = 1.0 - 2.0 * s_pgm / (s_pm + s_gm + _EPS)
    pos = s_gm
    neg = jnp.minimum(s_m - s_gm, _NEG_RATIO * pos)
    balanced = loss * (pos + neg) / (pos + neg + _EPS)
    fallback = loss * pos / (pos + _EPS)
    out_ref[0, 0] = jnp.where(neg > 0.0, balanced, fallback)


@jax.jit
def kernel(pred, gt, mask):
    partials = _sc_partials(
        pred.reshape(_ROWS, _COLS),
        gt.reshape(_ROWS, _COLS),
        mask.reshape(_ROWS, _COLS),
    )
    out = pl.pallas_call(
        _epilogue_body,
        out_specs=pl.BlockSpec((1, 1), memory_space=pltpu.SMEM),
        out_shape=jax.ShapeDtypeStruct((1, 1), jnp.float32),
    )(partials)
    return out.reshape(())


# speedup vs baseline: 2.4962x; 1.0427x over previous
import functools

import jax
import jax.numpy as jnp
from jax import lax
from jax.experimental import pallas as pl
from jax.experimental.pallas import tpu as pltpu
from jax.experimental.pallas import tpu_sc as plsc

_EPS = 1e-07
_NEG_RATIO = 3.0

_N = 16 * 512 * 512
_NC = 2
_NS = 16
_NW = _NC * _NS
_COLS = 512
_ROWS = _N // _COLS
_ROWS_W = _ROWS // _NW
_BAND = 32
_NCHUNKS = _ROWS_W // _BAND
_LANES = 16


def _sc_partials(pred, gt, mask):
    mesh = plsc.VectorSubcoreMesh(core_axis_name="c", subcore_axis_name="s")
    buf = pltpu.VMEM((_BAND, _COLS), jnp.float32)

    @functools.partial(
        pl.kernel,
        mesh=mesh,
        out_type=jax.ShapeDtypeStruct((_NW, 4 * _LANES), jnp.float32),
        scratch_types=[buf] * 6
        + [pltpu.SemaphoreType.DMA] * 6
        + [pltpu.VMEM((4 * _LANES,), jnp.float32)],
    )
    def body(p_hbm, g_hbm, m_hbm, out_hbm,
             pb0, gb0, mb0, pb1, gb1, mb1,
             sp0, sg0, sm0, sp1, sg1, sm1, acc):
        wid = lax.axis_index("s") * _NC + lax.axis_index("c")
        base = wid * _ROWS_W
        hbm = (p_hbm, g_hbm, m_hbm)
        bufs = ((pb0, gb0, mb0), (pb1, gb1, mb1))
        sems = ((sp0, sg0, sm0), (sp1, sg1, sm1))

        def start(ci):
            row0 = base + ci * _BAND
            slot = ci % 2
            return [
                pltpu.async_copy(h.at[pl.ds(row0, _BAND)], b, s)
                for h, b, s in zip(hbm, bufs[slot], sems[slot])
            ]

        _SUB = 4

        def compute(slot, accs):
            pb, gb, mb = bufs[slot]

            def inner(r, accs2):
                accl = list(accs2)
                for cu in range(_COLS // _LANES):
                    c = cu * _LANES
                    k = cu % _SUB
                    p = pb[r, pl.ds(c, _LANES)]
                    g = gb[r, pl.ds(c, _LANES)]
                    m = mb[r, pl.ds(c, _LANES)]
                    pm = p * m
                    accl[k] = accl[k] + m
                    accl[_SUB + k] = accl[_SUB + k] + g * m
                    accl[2 * _SUB + k] = accl[2 * _SUB + k] + pm
                    accl[3 * _SUB + k] = accl[3 * _SUB + k] + pm * g
                return tuple(accl)

            return lax.fori_loop(0, _BAND, inner, accs, unroll=False)

        z = jnp.zeros((_LANES,), jnp.float32)
        accs = (z,) * (4 * _SUB)
        pending = start(0)
        for ci in range(_NCHUNKS):
            nxt = start(ci + 1) if ci + 1 < _NCHUNKS else None
            for d in pending:
                d.wait()
            accs = compute(ci % 2, accs)
            pending = nxt
        sums = [
            functools.reduce(
                lambda a, b: a + b, accs[i * _SUB : (i + 1) * _SUB]
            )
            for i in range(4)
        ]
        a_m, a_gm, a_pm, a_pgm = sums
        acc[pl.ds(0, _LANES)] = a_m
        acc[pl.ds(_LANES, _LANES)] = a_gm
        acc[pl.ds(2 * _LANES, _LANES)] = a_pm
        acc[pl.ds(3 * _LANES, _LANES)] = a_pgm
        pltpu.sync_copy(acc, out_hbm.at[wid])

    return body(pred, gt, mask)


def _epilogue_body(part_ref, out_ref):
    x = part_ref[...]
    s_m = jnp.sum(x[:, 0:16])
    s_gm = jnp.sum(x[:, 16:32])
    s_pm = jnp.sum(x[:, 32:48])
    s_pgm = jnp.sum(x[:, 48:64])
    loss = 1.0 - 2.0 * s_pgm / (s_pm + s_gm + _EPS)
    pos = s_gm
    neg = jnp.minimum(s_m - s_gm, _NEG_RATIO * pos)
    balanced = loss * (pos + neg) / (pos + neg + _EPS)
    fallback = loss * pos / (pos + _EPS)
    out_ref[0, 0] = jnp.where(neg > 0.0, balanced, fallback)


@jax.jit
def kernel(pred, gt, mask):
    partials = _sc_partials(
        pred.reshape(_ROWS, _COLS),
        gt.reshape(_ROWS, _COLS),
        mask.reshape(_ROWS, _COLS),
    )
    out = pl.pallas_call(
        _epilogue_body,
        out_specs=pl.BlockSpec((1, 1), memory_space=pltpu.SMEM),
        out_shape=jax.ShapeDtypeStruct((1, 1), jnp.float32),
    )(partials)
    return out.reshape(())


# speedup vs baseline: 2.6129x; 1.0468x over previous
import functools

import jax
import jax.numpy as jnp
from jax import lax
from jax.experimental import pallas as pl
from jax.experimental.pallas import tpu as pltpu
from jax.experimental.pallas import tpu_sc as plsc

_EPS = 1e-07
_NEG_RATIO = 3.0

_N = 16 * 512 * 512
_NC = 2
_NS = 16
_NW = _NC * _NS
_COLS = 512
_ROWS = _N // _COLS
_ROWS_W = _ROWS // _NW
_BAND = 16
_NCHUNKS = _ROWS_W // _BAND
_RING = 4
_LANES = 16


def _sc_partials(pred, gt, mask):
    mesh = plsc.VectorSubcoreMesh(core_axis_name="c", subcore_axis_name="s")
    buf = pltpu.VMEM((_BAND, _COLS), jnp.float32)

    @functools.partial(
        pl.kernel,
        mesh=mesh,
        out_type=jax.ShapeDtypeStruct((_NW, 4 * _LANES), jnp.float32),
        scratch_types=[buf] * (3 * _RING)
        + [pltpu.SemaphoreType.DMA] * (3 * _RING)
        + [pltpu.VMEM((4 * _LANES,), jnp.float32)],
    )
    def body(p_hbm, g_hbm, m_hbm, out_hbm, *scr):
        wid = lax.axis_index("s") * _NC + lax.axis_index("c")
        base = wid * _ROWS_W
        hbm = (p_hbm, g_hbm, m_hbm)
        bufs = [scr[3 * i : 3 * i + 3] for i in range(_RING)]
        sems = [
            scr[3 * _RING + 3 * i : 3 * _RING + 3 * i + 3]
            for i in range(_RING)
        ]
        acc = scr[6 * _RING]
        last_row = base + (_NCHUNKS - 1) * _BAND

        def start(ci, slot):
            row0 = jnp.minimum(base + ci * _BAND, last_row)
            return [
                pltpu.async_copy(h.at[pl.ds(row0, _BAND)], b, s)
                for h, b, s in zip(hbm, bufs[slot], sems[slot])
            ]

        def wait(slot):
            for h, b, s in zip(hbm, bufs[slot], sems[slot]):
                pltpu.make_async_copy(h.at[pl.ds(0, _BAND)], b, s).wait()

        _SUB = 4

        def compute(slot, accs):
            pb, gb, mb = bufs[slot]

            def inner(r, accs2):
                accl = list(accs2)
                for cu in range(_COLS // _LANES):
                    c = cu * _LANES
                    k = cu % _SUB
                    p = pb[r, pl.ds(c, _LANES)]
                    g = gb[r, pl.ds(c, _LANES)]
                    m = mb[r, pl.ds(c, _LANES)]
                    pm = p * m
                    accl[k] = accl[k] + m
                    accl[_SUB + k] = accl[_SUB + k] + g * m
                    accl[2 * _SUB + k] = accl[2 * _SUB + k] + pm
                    accl[3 * _SUB + k] = accl[3 * _SUB + k] + pm * g
                return tuple(accl)

            return lax.fori_loop(0, _BAND, inner, accs, unroll=False)

        z = jnp.zeros((_LANES,), jnp.float32)
        for slot in range(_RING):
            start(slot, slot)

        def super_iter(t, accs):
            ci0 = t * _RING
            for b in range(_RING):
                wait(b)
                accs = compute(b, accs)
                start(ci0 + b + _RING, b)
            return accs

        accs = lax.fori_loop(
            0, _NCHUNKS // _RING, super_iter, (z,) * (4 * _SUB)
        )
        for slot in range(_RING):
            wait(slot)
        sums = [
            functools.reduce(
                lambda a, b: a + b, accs[i * _SUB : (i + 1) * _SUB]
            )
            for i in range(4)
        ]
        a_m, a_gm, a_pm, a_pgm = sums
        acc[pl.ds(0, _LANES)] = a_m
        acc[pl.ds(_LANES, _LANES)] = a_gm
        acc[pl.ds(2 * _LANES, _LANES)] = a_pm
        acc[pl.ds(3 * _LANES, _LANES)] = a_pgm
        pltpu.sync_copy(acc, out_hbm.at[wid])

    return body(pred, gt, mask)


def _epilogue_body(part_ref, out_ref):
    x = part_ref[...]
    s_m = jnp.sum(x[:, 0:16])
    s_gm = jnp.sum(x[:, 16:32])
    s_pm = jnp.sum(x[:, 32:48])
    s_pgm = jnp.sum(x[:, 48:64])
    loss = 1.0 - 2.0 * s_pgm / (s_pm + s_gm + _EPS)
    pos = s_gm
    neg = jnp.minimum(s_m - s_gm, _NEG_RATIO * pos)
    balanced = loss * (pos + neg) / (pos + neg + _EPS)
    fallback = loss * pos / (pos + _EPS)
    out_ref[0, 0] = jnp.where(neg > 0.0, balanced, fallback)


@jax.jit
def kernel(pred, gt, mask):
    partials = _sc_partials(
        pred.reshape(_ROWS, _COLS),
        gt.reshape(_ROWS, _COLS),
        mask.reshape(_ROWS, _COLS),
    )
    out = pl.pallas_call(
        _epilogue_body,
        out_specs=pl.BlockSpec((1, 1), memory_space=pltpu.SMEM),
        out_shape=jax.ShapeDtypeStruct((1, 1), jnp.float32),
    )(partials)
    return out.reshape(())


# speedup vs baseline: 3.0768x; 1.1776x over previous
import functools

import jax
import jax.numpy as jnp
from jax import lax
from jax.experimental import pallas as pl
from jax.experimental.pallas import tpu as pltpu
from jax.experimental.pallas import tpu_sc as plsc

_EPS = 1e-07
_NEG_RATIO = 3.0

_N = 16 * 512 * 512
_NC = 2
_NS = 16
_NW = _NC * _NS
_COLS = 512
_ROWS = _N // _COLS
_SC_ROWS = 4096
_ROWS_W = _SC_ROWS // _NW
_BAND = 16
_NCHUNKS = _ROWS_W // _BAND
_RING = 4
_LANES = 16
_TC_BLOCK = 512


def _sc_partials(pred, gt, mask):
    mesh = plsc.VectorSubcoreMesh(core_axis_name="c", subcore_axis_name="s")
    buf = pltpu.VMEM((_BAND, _COLS), jnp.float32)

    @functools.partial(
        pl.kernel,
        mesh=mesh,
        out_type=jax.ShapeDtypeStruct((_NW, 4 * _LANES), jnp.float32),
        scratch_types=[buf] * (3 * _RING)
        + [pltpu.SemaphoreType.DMA] * (3 * _RING)
        + [pltpu.VMEM((4 * _LANES,), jnp.float32)],
    )
    def body(p_hbm, g_hbm, m_hbm, out_hbm, *scr):
        wid = lax.axis_index("s") * _NC + lax.axis_index("c")
        base = wid * _ROWS_W
        hbm = (p_hbm, g_hbm, m_hbm)
        bufs = [scr[3 * i : 3 * i + 3] for i in range(_RING)]
        sems = [
            scr[3 * _RING + 3 * i : 3 * _RING + 3 * i + 3]
            for i in range(_RING)
        ]
        acc = scr[6 * _RING]
        last_row = base + (_NCHUNKS - 1) * _BAND

        def start(ci, slot):
            row0 = jnp.minimum(base + ci * _BAND, last_row)
            return [
                pltpu.async_copy(h.at[pl.ds(row0, _BAND)], b, s)
                for h, b, s in zip(hbm, bufs[slot], sems[slot])
            ]

        def wait(slot):
            for h, b, s in zip(hbm, bufs[slot], sems[slot]):
                pltpu.make_async_copy(h.at[pl.ds(0, _BAND)], b, s).wait()

        _SUB = 4

        def compute(slot, accs):
            pb, gb, mb = bufs[slot]

            def inner(r, accs2):
                accl = list(accs2)
                for cu in range(_COLS // _LANES):
                    c = cu * _LANES
                    k = cu % _SUB
                    p = pb[r, pl.ds(c, _LANES)]
                    g = gb[r, pl.ds(c, _LANES)]
                    m = mb[r, pl.ds(c, _LANES)]
                    pm = p * m
                    accl[k] = accl[k] + m
                    accl[_SUB + k] = accl[_SUB + k] + g * m
                    accl[2 * _SUB + k] = accl[2 * _SUB + k] + pm
                    accl[3 * _SUB + k] = accl[3 * _SUB + k] + pm * g
                return tuple(accl)

            return lax.fori_loop(0, _BAND, inner, accs, unroll=False)

        z = jnp.zeros((_LANES,), jnp.float32)
        for slot in range(_RING):
            start(slot, slot)

        def super_iter(t, accs):
            ci0 = t * _RING
            for b in range(_RING):
                wait(b)
                accs = compute(b, accs)
                start(ci0 + b + _RING, b)
            return accs

        accs = lax.fori_loop(
            0, _NCHUNKS // _RING, super_iter, (z,) * (4 * _SUB)
        )
        for slot in range(_RING):
            wait(slot)
        sums = [
            functools.reduce(
                lambda a, b: a + b, accs[i * _SUB : (i + 1) * _SUB]
            )
            for i in range(4)
        ]
        a_m, a_gm, a_pm, a_pgm = sums
        acc[pl.ds(0, _LANES)] = a_m
        acc[pl.ds(_LANES, _LANES)] = a_gm
        acc[pl.ds(2 * _LANES, _LANES)] = a_pm
        acc[pl.ds(3 * _LANES, _LANES)] = a_pgm
        pltpu.sync_copy(acc, out_hbm.at[wid])

    return body(pred, gt, mask)


def _tc_body(p_ref, g_ref, m_ref, out_ref):
    i = pl.program_id(0)

    @pl.when(i == 0)
    def _init():
        out_ref[...] = jnp.zeros_like(out_ref)

    p = p_ref[...]
    g = g_ref[...]
    m = m_ref[...]
    pm = p * m
    out_ref[0, :] = out_ref[0, :] + jnp.sum(m, axis=0)
    out_ref[1, :] = out_ref[1, :] + jnp.sum(g * m, axis=0)
    out_ref[2, :] = out_ref[2, :] + jnp.sum(pm, axis=0)
    out_ref[3, :] = out_ref[3, :] + jnp.sum(pm * g, axis=0)


def _tc_partials(p2, g2, m2):
    steps = (_ROWS - _SC_ROWS) // _TC_BLOCK
    off = _SC_ROWS // _TC_BLOCK
    in_spec = pl.BlockSpec((_TC_BLOCK, _COLS), lambda i: (off + i, 0))
    return pl.pallas_call(
        _tc_body,
        grid=(steps,),
        in_specs=[in_spec, in_spec, in_spec],
        out_specs=pl.BlockSpec((4, _COLS), lambda i: (0, 0)),
        out_shape=jax.ShapeDtypeStruct((4, _COLS), jnp.float32),
    )(p2, g2, m2)


def _epilogue_body(part_ref, tc_ref, out_ref):
    x = part_ref[...]
    t = tc_ref[...]
    s_m = jnp.sum(x[:, 0:16]) + jnp.sum(t[0, :])
    s_gm = jnp.sum(x[:, 16:32]) + jnp.sum(t[1, :])
    s_pm = jnp.sum(x[:, 32:48]) + jnp.sum(t[2, :])
    s_pgm = jnp.sum(x[:, 48:64]) + jnp.sum(t[3, :])
    loss = 1.0 - 2.0 * s_pgm / (s_pm + s_gm + _EPS)
    pos = s_gm
    neg = jnp.minimum(s_m - s_gm, _NEG_RATIO * pos)
    balanced = loss * (pos + neg) / (pos + neg + _EPS)
    fallback = loss * pos / (pos + _EPS)
    out_ref[0, 0] = jnp.where(neg > 0.0, balanced, fallback)


@jax.jit
def kernel(pred, gt, mask):
    p2 = pred.reshape(_ROWS, _COLS)
    g2 = gt.reshape(_ROWS, _COLS)
    m2 = mask.reshape(_ROWS, _COLS)
    sc_part = _sc_partials(p2, g2, m2)
    tc_part = _tc_partials(p2, g2, m2)
    out = pl.pallas_call(
        _epilogue_body,
        out_specs=pl.BlockSpec((1, 1), memory_space=pltpu.SMEM),
        out_shape=jax.ShapeDtypeStruct((1, 1), jnp.float32),
    )(sc_part, tc_part)
    return out.reshape(())
